# baseline (speedup 1.0000x reference)
import functools

import jax
import jax.numpy as jnp
from jax import lax
from jax.experimental import pallas as pl
from jax.experimental.pallas import tpu as pltpu
from jax.experimental.pallas import tpu_sc as plsc

BATCH = 16384
D = 32
WIDE = 128
NC = 2
NS = 16
L = 16
NW = NC * NS
BPW = BATCH // NW
CHUNK = 128
NCHUNK = BPW // CHUNK
GPC = CHUNK // L


_mesh = plsc.VectorSubcoreMesh(
    core_axis_name="c", subcore_axis_name="s", num_cores=NC, num_subcores=NS
)


@functools.partial(
    pl.kernel,
    out_type=jax.ShapeDtypeStruct((BATCH,), jnp.float32),
    mesh=_mesh,
    compiler_params=pltpu.CompilerParams(
        needs_layout_passes=False, use_tc_tiling_on_sc=False
    ),
    scratch_types=[
        pltpu.VMEM((NCHUNK, CHUNK), jnp.int32),
        pltpu.VMEM((NCHUNK, CHUNK), jnp.int32),
        pltpu.VMEM((NCHUNK, CHUNK), jnp.int32),
        pltpu.VMEM((NCHUNK, CHUNK), jnp.int32),
        pltpu.VMEM((2, CHUNK, WIDE), jnp.float32),
        pltpu.VMEM((2, CHUNK, WIDE), jnp.float32),
        pltpu.VMEM((BPW,), jnp.float32),
        pltpu.SemaphoreType.DMA((2,)),
    ],
)
def _mf_kernel(user_hbm, item_hbm, ut_hbm, it_hbm, out_hbm,
               uidx, iidx, umaj, imaj, ubuf, ibuf, outv, sem):
    wid = lax.axis_index("s") * NC + lax.axis_index("c")
    base = wid * BPW

    pltpu.sync_copy(user_hbm.at[pl.ds(wid * NCHUNK, NCHUNK)], uidx)
    pltpu.sync_copy(item_hbm.at[pl.ds(wid * NCHUNK, NCHUNK)], iidx)

    for j in range(NCHUNK):
        for v in range(GPC):
            sl = pl.ds(v * L, L)
            umaj[j, sl] = lax.shift_right_logical(uidx[j, sl], 2)
            imaj[j, sl] = lax.shift_right_logical(iidx[j, sl], 2)

    def fire(j, p):
        cu = pltpu.make_async_copy(ut_hbm.at[umaj.at[j]], ubuf.at[p], sem.at[p])
        ci = pltpu.make_async_copy(it_hbm.at[imaj.at[j]], ibuf.at[p], sem.at[p])
        return cu, ci

    cu0, ci0 = fire(0, 0)
    cu0.start()
    ci0.start()

    lane = lax.iota(jnp.int32, L)

    def chunk_body(j, carry):
        p = jnp.bitwise_and(j, 1)
        pn = jnp.bitwise_and(j + 1, 1)

        @pl.when(j < NCHUNK - 1)
        def _():
            cu, ci = fire(j + 1, pn)
            cu.start()
            ci.start()

        cu, ci = fire(j, p)
        cu.wait()
        ci.wait()

        pv = jnp.full((L,), p, jnp.int32)

        def group_body(g, carry2):
            sl = pl.ds(g * L, L)
            rows = g * L + lane
            ucol0 = lax.shift_left(jnp.bitwise_and(uidx[j, sl], 3), 5)
            icol0 = lax.shift_left(jnp.bitwise_and(iidx[j, sl], 3), 5)
            acc = jnp.zeros((L,), jnp.float32)
            for d in range(D):
                acc = acc + (plsc.load_gather(ubuf, [pv, rows, ucol0 + d])
                             * plsc.load_gather(ibuf, [pv, rows, icol0 + d]))
            outv[pl.ds(j * CHUNK + g * L, L)] = acc
            return carry2

        lax.fori_loop(0, GPC, group_body, 0)
        return carry

    lax.fori_loop(0, NCHUNK, chunk_body, 0)

    pltpu.sync_copy(outv, out_hbm.at[pl.ds(base, BPW)])


def kernel(user, item, user_table, item_table):
    user2d = user.reshape(NW * NCHUNK, CHUNK)
    item2d = item.reshape(NW * NCHUNK, CHUNK)
    ut_wide = user_table.reshape(-1, WIDE)
    it_wide = item_table.reshape(-1, WIDE)
    return _mf_kernel(user2d, item2d, ut_wide, it_wide)


# speedup vs baseline: 1.0002x; 1.0002x over previous
import functools

import jax
import jax.numpy as jnp
from jax import lax
from jax.experimental import pallas as pl
from jax.experimental.pallas import tpu as pltpu
from jax.experimental.pallas import tpu_sc as plsc

BATCH = 16384
D = 32
WIDE = 128
NC = 2
NS = 16
L = 16
NW = NC * NS
BPW = BATCH // NW
CHUNK = 128
NCHUNK = BPW // CHUNK
GPC = CHUNK // L


_mesh = plsc.VectorSubcoreMesh(
    core_axis_name="c", subcore_axis_name="s", num_cores=NC, num_subcores=NS
)


@functools.partial(
    pl.kernel,
    out_type=jax.ShapeDtypeStruct((BATCH,), jnp.float32),
    mesh=_mesh,
    compiler_params=pltpu.CompilerParams(
        needs_layout_passes=False, use_tc_tiling_on_sc=True
    ),
    scratch_types=[
        pltpu.VMEM((NCHUNK, CHUNK), jnp.int32),
        pltpu.VMEM((NCHUNK, CHUNK), jnp.int32),
        pltpu.VMEM((NCHUNK, CHUNK), jnp.int32),
        pltpu.VMEM((NCHUNK, CHUNK), jnp.int32),
        pltpu.VMEM((2, CHUNK, WIDE), jnp.float32),
        pltpu.VMEM((2, CHUNK, WIDE), jnp.float32),
        pltpu.VMEM((BPW,), jnp.float32),
        pltpu.SemaphoreType.DMA((2,)),
    ],
)
def _mf_kernel(user_hbm, item_hbm, ut_hbm, it_hbm, out_hbm,
               uidx, iidx, umaj, imaj, ubuf, ibuf, outv, sem):
    wid = lax.axis_index("s") * NC + lax.axis_index("c")
    base = wid * BPW

    pltpu.sync_copy(user_hbm.at[pl.ds(wid * NCHUNK, NCHUNK)], uidx)
    pltpu.sync_copy(item_hbm.at[pl.ds(wid * NCHUNK, NCHUNK)], iidx)

    for j in range(NCHUNK):
        for v in range(GPC):
            sl = pl.ds(v * L, L)
            umaj[j, sl] = lax.shift_right_logical(uidx[j, sl], 2)
            imaj[j, sl] = lax.shift_right_logical(iidx[j, sl], 2)

    def fire(j, p):
        cu = pltpu.make_async_copy(ut_hbm.at[umaj.at[j]], ubuf.at[p], sem.at[p])
        ci = pltpu.make_async_copy(it_hbm.at[imaj.at[j]], ibuf.at[p], sem.at[p])
        return cu, ci

    cu0, ci0 = fire(0, 0)
    cu0.start()
    ci0.start()

    lane = lax.iota(jnp.int32, L)

    def chunk_body(j, carry):
        p = jnp.bitwise_and(j, 1)
        pn = jnp.bitwise_and(j + 1, 1)

        @pl.when(j < NCHUNK - 1)
        def _():
            cu, ci = fire(j + 1, pn)
            cu.start()
            ci.start()

        cu, ci = fire(j, p)
        cu.wait()
        ci.wait()

        pv = jnp.full((L,), p, jnp.int32)

        def group_body(g, carry2):
            sl = pl.ds(g * L, L)
            rows = g * L + lane
            ucol0 = lax.shift_left(jnp.bitwise_and(uidx[j, sl], 3), 5)
            icol0 = lax.shift_left(jnp.bitwise_and(iidx[j, sl], 3), 5)
            acc = jnp.zeros((L,), jnp.float32)
            for d in range(D):
                acc = acc + (plsc.load_gather(ubuf, [pv, rows, ucol0 + d])
                             * plsc.load_gather(ibuf, [pv, rows, icol0 + d]))
            outv[pl.ds(j * CHUNK + g * L, L)] = acc
            return carry2

        lax.fori_loop(0, GPC, group_body, 0)
        return carry

    lax.fori_loop(0, NCHUNK, chunk_body, 0)

    pltpu.sync_copy(outv, out_hbm.at[pl.ds(base, BPW)])


def kernel(user, item, user_table, item_table):
    user2d = user.reshape(NW * NCHUNK, CHUNK)
    item2d = item.reshape(NW * NCHUNK, CHUNK)
    ut_wide = user_table.reshape(-1, WIDE)
    it_wide = item_table.reshape(-1, WIDE)
    return _mf_kernel(user2d, item2d, ut_wide, it_wide)


# speedup vs baseline: 3.8602x; 3.8593x over previous
import functools

import jax
import jax.numpy as jnp
from jax import lax
from jax.experimental import pallas as pl
from jax.experimental.pallas import tpu as pltpu
from jax.experimental.pallas import tpu_sc as plsc

BATCH = 16384
D = 32
NC = 2
NS = 16
L = 16
NW = NC * NS
BPW = BATCH // NW
NG = BPW // L
RING = 8
WID = 128


_mesh = plsc.VectorSubcoreMesh(
    core_axis_name="c", subcore_axis_name="s", num_cores=NC, num_subcores=NS
)


@functools.partial(
    pl.kernel,
    out_type=jax.ShapeDtypeStruct((BATCH,), jnp.float32),
    mesh=_mesh,
    compiler_params=pltpu.CompilerParams(
        needs_layout_passes=False, use_tc_tiling_on_sc=True
    ),
    scratch_types=[
        pltpu.VMEM((BPW,), jnp.int32),
        pltpu.VMEM((BPW,), jnp.int32),
        pltpu.VMEM((RING, D, WID), jnp.float32),
        pltpu.VMEM((RING, D, WID), jnp.float32),
        pltpu.VMEM((BPW,), jnp.float32),
        pltpu.SemaphoreType.DMA((RING,)),
    ],
)
def _mf_kernel(user_hbm, item_hbm, utT_hbm, itT_hbm, out_hbm,
               uidx, iidx, ubuf, ibuf, outv, sem):
    wid = lax.axis_index("s") * NC + lax.axis_index("c")
    base = wid * BPW

    pltpu.sync_copy(user_hbm.at[pl.ds(base, BPW)], uidx)
    pltpu.sync_copy(item_hbm.at[pl.ds(base, BPW)], iidx)

    lane = lax.iota(jnp.int32, L)
    dlo = lane
    dhi = lane + L

    def fire(uv, iv, k, slot):
        ru = jnp.bitwise_and(uv[k], ~127)
        ri = jnp.bitwise_and(iv[k], ~127)
        pltpu.async_copy(utT_hbm.at[:, pl.ds(pl.multiple_of(ru, WID), WID)],
                         ubuf.at[slot], sem.at[slot])
        pltpu.async_copy(itT_hbm.at[:, pl.ds(pl.multiple_of(ri, WID), WID)],
                         ibuf.at[slot], sem.at[slot])

    def drain_compute(uv, iv, k, slot, acc):
        ru = jnp.bitwise_and(uv[k], ~127)
        ri = jnp.bitwise_and(iv[k], ~127)
        pltpu.make_async_copy(
            utT_hbm.at[:, pl.ds(pl.multiple_of(ru, WID), WID)],
            ubuf.at[slot], sem.at[slot]).wait()
        pltpu.make_async_copy(
            itT_hbm.at[:, pl.ds(pl.multiple_of(ri, WID), WID)],
            ibuf.at[slot], sem.at[slot]).wait()
        sv = jnp.full((L,), slot, jnp.int32)
        lu = jnp.full((L,), jnp.bitwise_and(uv[k], 127), jnp.int32)
        li = jnp.full((L,), jnp.bitwise_and(iv[k], 127), jnp.int32)
        p = (plsc.load_gather(ubuf, [sv, dlo, lu])
             * plsc.load_gather(ibuf, [sv, dlo, li])
             + plsc.load_gather(ubuf, [sv, dhi, lu])
             * plsc.load_gather(ibuf, [sv, dhi, li]))
        s = jnp.sum(p)
        return jnp.where(lane == k, s, acc)

    uv0 = uidx[pl.ds(0, L)]
    iv0 = iidx[pl.ds(0, L)]
    for e in range(RING):
        fire(uv0, iv0, e, e)

    def group_body(g, carry):
        uvp, ivp, acc = carry
        uv = uidx[pl.ds(g * L, L)]
        iv = iidx[pl.ds(g * L, L)]

        @pl.when(g > 0)
        def _():
            a = acc
            for e in range(RING):
                a = drain_compute(uvp, ivp, RING + e, e, a)
            outv[pl.ds((g - 1) * L, L)] = a

        @pl.when(g > 0)
        def _():
            for e in range(RING):
                fire(uv, iv, e, e)

        acc2 = jnp.zeros((L,), jnp.float32)
        for e in range(RING):
            acc2 = drain_compute(uv, iv, e, e, acc2)
            fire(uv, iv, RING + e, e)

        return uv, iv, acc2

    uvl, ivl, accl = lax.fori_loop(
        0, NG, group_body,
        (uv0, iv0, jnp.zeros((L,), jnp.float32)))

    for e in range(RING):
        accl = drain_compute(uvl, ivl, RING + e, e, accl)
    outv[pl.ds((NG - 1) * L, L)] = accl

    pltpu.sync_copy(outv, out_hbm.at[pl.ds(base, BPW)])


def kernel(user, item, user_table, item_table):
    return _mf_kernel(user, item, user_table.T, item_table.T)
